# baseline (speedup 1.0000x reference)
import functools

import jax
import jax.numpy as jnp
from jax import lax
from jax.experimental import pallas as pl
from jax.experimental.pallas import tpu as pltpu
from jax.experimental.pallas import tpu_sc as plsc

VOCAB = 1000000
NUM_CORES = 2
NUM_SUBCORES = 16
NUM_WORKERS = NUM_CORES * NUM_SUBCORES
LANES = 16
CHUNK = 128


def _stay_body(th_ref, tn_ref, dh_ref, dn_ref, out_ref):
    cond = dh_ref[...] != dn_ref[...]
    out_ref[...] = jnp.where(cond, 48, 0) + tn_ref[...] - th_ref[...]


def _stay_time(ts_his, ts_next, day_his, day_next):
    n_rows, n_cols = ts_his.shape
    block = 512
    grid = n_rows // block
    spec = pl.BlockSpec((block, n_cols), lambda i: (i, 0))
    return pl.pallas_call(
        _stay_body,
        grid=(grid,),
        in_specs=[spec, spec, spec, spec],
        out_specs=spec,
        out_shape=jax.ShapeDtypeStruct((n_rows, n_cols), jnp.int32),
    )(ts_his, ts_next, day_his, day_next)


def _make_gather(n_idx, embed_dim):
    per_w = n_idx // NUM_WORKERS
    n_chunks = per_w // CHUNK
    mesh = plsc.VectorSubcoreMesh(
        core_axis_name="c",
        subcore_axis_name="s",
        num_cores=NUM_CORES,
        num_subcores=NUM_SUBCORES,
    )

    @functools.partial(
        pl.kernel,
        out_type=jax.ShapeDtypeStruct((n_idx, embed_dim), jnp.float32),
        mesh=mesh,
        compiler_params=pltpu.CompilerParams(use_tc_tiling_on_sc=False),
        scratch_types=[
            pltpu.VMEM((per_w,), jnp.int32),
            pltpu.VMEM((per_w,), jnp.int32),
            pltpu.VMEM((2, CHUNK, embed_dim), jnp.float32),
            pltpu.SemaphoreType.DMA,
            pltpu.SemaphoreType.DMA,
        ],
    )
    def gather_kernel(table_hbm, x_hbm, city_hbm, out_hbm, idx_v, city_v,
                      rows_v, gsem, ssem):
        wid = lax.axis_index("c") * NUM_SUBCORES + lax.axis_index("s")
        wbase = wid * per_w
        pltpu.sync_copy(x_hbm.at[pl.ds(wbase, per_w)], idx_v)
        pltpu.sync_copy(city_hbm.at[pl.ds(wbase, per_w)], city_v)

        def idx_chunk(ci):
            for j in range(CHUNK // LANES):
                o = ci * CHUNK + j * LANES
                s = idx_v[pl.ds(o, LANES)] + city_v[pl.ds(o, LANES)]
                idx_v[pl.ds(o, LANES)] = jnp.where(s >= VOCAB, s - VOCAB, s)

        def body(ci, carry):
            idx_chunk(ci)
            base = ci * CHUNK
            pltpu.async_copy(
                table_hbm.at[idx_v.at[pl.ds(base, CHUNK)]],
                rows_v.at[0],
                gsem,
            ).wait()
            pltpu.sync_copy(rows_v.at[0], out_hbm.at[pl.ds(wbase + base, CHUNK)])
            return carry

        lax.fori_loop(0, n_chunks, body, 0)

    return gather_kernel


def kernel(traj, vocab_tot):
    batch, hist_p1, _ = traj.shape
    his_len = hist_p1 - 1
    t = traj.astype(jnp.int32)
    x_test = t[:, :-1, 0]
    y_test = t[:, 1:, 0]
    ts_his = t[:, :-1, 1]
    ts_next = t[:, 1:, 1]
    day_his = t[:, :-1, 2]
    day_next = t[:, 1:, 2]
    test_city = t[:, :-1, 3]

    stay_time = _stay_time(ts_his, ts_next, day_his, day_next)

    n_idx = batch * his_len
    gather = _make_gather(n_idx, vocab_tot.shape[1])
    poi = gather(vocab_tot, x_test.reshape(n_idx), test_city.reshape(n_idx))
    return poi.reshape(batch, his_len, vocab_tot.shape[1]), stay_time, y_test
